# baseline (speedup 1.0000x reference)
import functools

import jax
import jax.numpy as jnp
from jax import lax
from jax.experimental import pallas as pl
from jax.experimental.pallas import tpu as pltpu
from jax.experimental.pallas import tpu_sc as plsc

ROWS = 1000001
EMBED = 32
BATCH = 16384
_CHUNK = 512
_NCHUNK = BATCH // _CHUNK
_NBUF = 4

_info = plsc.get_sparse_core_info()
_NC, _NS = _info.num_cores, _info.num_subcores

_mesh = plsc.VectorSubcoreMesh(core_axis_name="c", subcore_axis_name="s")


@functools.partial(
    pl.kernel,
    mesh=_mesh,
    compiler_params=pltpu.CompilerParams(
        use_tc_tiling_on_sc=False, needs_layout_passes=False
    ),
    out_type=jax.ShapeDtypeStruct((EMBED, BATCH), jnp.float32),
    scratch_types=[
        pltpu.VMEM((BATCH,), jnp.int32),
        pltpu.VMEM((BATCH,), jnp.int32),
        pltpu.VMEM((BATCH,), jnp.int32),
        pltpu.VMEM((_NBUF, _CHUNK, 8), jnp.float32),
        pltpu.VMEM((BATCH,), jnp.float32),
        pltpu.SemaphoreType.DMA((_NBUF,)),
    ],
)
def _gather_kernel(idx_hbm, tbl_hbm, out_hbm, idx_v, q_v, t_v, blk_v, col_v, sems):
    c = lax.axis_index("s") * _NC + lax.axis_index("c")
    pltpu.sync_copy(idx_hbm, idx_v)
    cbase = c * ROWS

    def compute(i, carry):
        e = idx_v[pl.ds(i * 16, 16)] + cbase
        q_v[pl.ds(i * 16, 16)] = e >> 3
        t_v[pl.ds(i * 16, 16)] = e & 7
        return carry

    lax.fori_loop(0, BATCH // 16, compute, 0)

    def fire(k, j):
        pltpu.async_copy(
            tbl_hbm.at[q_v.at[pl.ds(k * _CHUNK, _CHUNK)]], blk_v.at[j], sems.at[j]
        )

    def select(k, j):
        base = k * _CHUNK

        def body(i, carry):
            row = jax.lax.iota(jnp.int32, 16) + i * 16
            vals = plsc.load_gather(blk_v.at[j], [row, t_v[pl.ds(base + i * 16, 16)]])
            col_v[pl.ds(base + i * 16, 16)] = vals
            return carry

        lax.fori_loop(0, _CHUNK // 16, body, 0)

    for k in range(_NCHUNK):
        j = k % _NBUF
        if k >= _NBUF:
            pltpu.make_async_copy(tbl_hbm.at[pl.ds(0, _CHUNK)], blk_v.at[j], sems.at[j]).wait()
            select(k - _NBUF, j)
        fire(k, j)
    for k in range(_NCHUNK - _NBUF, _NCHUNK):
        j = k % _NBUF
        pltpu.make_async_copy(tbl_hbm.at[pl.ds(0, _CHUNK)], blk_v.at[j], sems.at[j]).wait()
        select(k, j)

    pltpu.sync_copy(col_v, out_hbm.at[c])


def kernel(indices, table):
    tbl2 = jnp.ravel(table.T).reshape(EMBED * ROWS // 8, 8)
    out_t = _gather_kernel(indices.astype(jnp.int32), tbl2)
    return out_t.T


# speedup vs baseline: 18.2907x; 18.2907x over previous
import functools

import jax
import jax.numpy as jnp
from jax import lax
from jax.experimental import pallas as pl
from jax.experimental.pallas import tpu as pltpu
from jax.experimental.pallas import tpu_sc as plsc

ROWS = 1000001
PADROWS = 1000064
EMBED = 32
BATCH = 16384
_CW = 76928
_NK = PADROWS // _CW
_CHUNK = 512
_NCHUNK = BATCH // _CHUNK
_NBUF = 4

_info = plsc.get_sparse_core_info()
_NC, _NS = _info.num_cores, _info.num_subcores

_mesh = plsc.VectorSubcoreMesh(core_axis_name="c", subcore_axis_name="s")


def _copy_body(in_ref, out_ref):
    blk = in_ref[...].reshape(8, _CW // 128, 128)
    out_ref[...] = jnp.transpose(blk, (1, 0, 2)).reshape(_CW * 8)


_detile = pl.pallas_call(
    _copy_body,
    grid=(EMBED // 8, _NK),
    in_specs=[pl.BlockSpec((8, _CW), lambda g, k: (g, k))],
    out_specs=pl.BlockSpec((_CW * 8,), lambda g, k: (g * _NK + k,)),
    out_shape=jax.ShapeDtypeStruct((EMBED * PADROWS,), jnp.float32),
)


@functools.partial(
    pl.kernel,
    mesh=_mesh,
    compiler_params=pltpu.CompilerParams(
        use_tc_tiling_on_sc=False, needs_layout_passes=False
    ),
    out_type=jax.ShapeDtypeStruct((EMBED, BATCH), jnp.float32),
    scratch_types=[
        pltpu.VMEM((BATCH,), jnp.int32),
        pltpu.VMEM((BATCH,), jnp.int32),
        pltpu.VMEM((BATCH,), jnp.int32),
        pltpu.VMEM((_NBUF, _CHUNK, 8), jnp.float32),
        pltpu.VMEM((BATCH,), jnp.float32),
        pltpu.SemaphoreType.DMA((_NBUF,)),
    ],
)
def _gather_kernel(idx_hbm, tbl_hbm, out_hbm, idx_v, q_v, t_v, blk_v, col_v, sems):
    c = lax.axis_index("s") * _NC + lax.axis_index("c")
    pltpu.sync_copy(idx_hbm, idx_v)
    cbase = (c >> 3) * (PADROWS * 8) + (c & 7) * 128

    def compute(i, carry):
        r = idx_v[pl.ds(i * 16, 16)]
        e = (r >> 7) * 1024 + (r & 127) + cbase
        q_v[pl.ds(i * 16, 16)] = e >> 3
        t_v[pl.ds(i * 16, 16)] = e & 7
        return carry

    lax.fori_loop(0, BATCH // 16, compute, 0)

    def fire(k, j):
        pltpu.async_copy(
            tbl_hbm.at[q_v.at[pl.ds(k * _CHUNK, _CHUNK)]], blk_v.at[j], sems.at[j]
        )

    def select(k, j):
        base = k * _CHUNK

        def body(i, carry):
            row = jax.lax.iota(jnp.int32, 16) + i * 16
            vals = plsc.load_gather(blk_v.at[j], [row, t_v[pl.ds(base + i * 16, 16)]])
            col_v[pl.ds(base + i * 16, 16)] = vals
            return carry

        lax.fori_loop(0, _CHUNK // 16, body, 0)

    for k in range(_NCHUNK):
        j = k % _NBUF
        if k >= _NBUF:
            pltpu.make_async_copy(tbl_hbm.at[pl.ds(0, _CHUNK)], blk_v.at[j], sems.at[j]).wait()
            select(k - _NBUF, j)
        fire(k, j)
    for k in range(_NCHUNK - _NBUF, _NCHUNK):
        j = k % _NBUF
        pltpu.make_async_copy(tbl_hbm.at[pl.ds(0, _CHUNK)], blk_v.at[j], sems.at[j]).wait()
        select(k, j)

    pltpu.sync_copy(col_v, out_hbm.at[c])


def kernel(indices, table):
    flat = _detile(table.T)
    tbl2 = flat.reshape(EMBED * PADROWS // 8, 8)
    out_t = _gather_kernel(indices.astype(jnp.int32), tbl2)
    return out_t.T


# speedup vs baseline: 18.4225x; 1.0072x over previous
import functools

import jax
import jax.numpy as jnp
from jax import lax
from jax.experimental import pallas as pl
from jax.experimental.pallas import tpu as pltpu
from jax.experimental.pallas import tpu_sc as plsc

ROWS = 1000001
PADROWS = 1000064
RMAIN = 999936
EMBED = 32
BATCH = 16384
_CW = 76928
_NK = PADROWS // _CW
_CHUNK = 512
_NCHUNK = BATCH // _CHUNK
_NBUF = 4
_SCW = 24
_TPW = RMAIN // 128 // 12
_CCH = 31 * 128
_NCC = _TPW // 31

_info = plsc.get_sparse_core_info()
_NC, _NS = _info.num_cores, _info.num_subcores

_mesh = plsc.VectorSubcoreMesh(core_axis_name="c", subcore_axis_name="s")


def _copy_body(in_ref, out_ref):
    blk = in_ref[...].reshape(8, _CW // 128, 128)
    out_ref[...] = jnp.transpose(blk, (1, 0, 2)).reshape(_CW * 8)


_detile = pl.pallas_call(
    _copy_body,
    grid=(2, _NK),
    in_specs=[pl.BlockSpec((8, _CW), lambda g, k: (g, k))],
    out_specs=pl.BlockSpec((_CW * 8,), lambda g, k: (g * _NK + k,)),
    out_shape=jax.ShapeDtypeStruct((2 * 8 * PADROWS,), jnp.float32),
)


@functools.partial(
    pl.kernel,
    mesh=_mesh,
    out_type=jax.ShapeDtypeStruct((2, RMAIN // 128 + 1, 8, 128), jnp.float32),
    scratch_types=[
        pltpu.VMEM((2, 31, 8, 128), jnp.float32),
        pltpu.SemaphoreType.DMA((2,)),
        pltpu.SemaphoreType.DMA((2,)),
    ],
)
def _sc_copy(tbl3_hbm, out_hbm, buf_v, rsems, wsems):
    w = lax.axis_index("s") * _NC + lax.axis_index("c")

    @pl.when(w < _SCW)
    def _():
        slab = 2 + w // 12
        tile0 = (w % 12) * _TPW
        for i in range(_NCC):
            j = i % 2
            b0 = tile0 + i * 31
            if i >= 2:
                pltpu.make_async_copy(
                    tbl3_hbm.at[0, :, pl.ds(0, _CCH)], buf_v.at[j], wsems.at[j]
                ).wait()
            for b in range(31):
                pltpu.async_copy(
                    tbl3_hbm.at[slab, :, pl.ds((b0 + b) * 128, 128)],
                    buf_v.at[j, b],
                    rsems.at[j],
                )
            pltpu.make_async_copy(
                tbl3_hbm.at[0, :, pl.ds(0, _CCH)], buf_v.at[j], rsems.at[j]
            ).wait()
            pltpu.async_copy(
                buf_v.at[j], out_hbm.at[slab - 2, pl.ds(b0, 31)], wsems.at[j]
            )
        for i in range(_NCC - 2, _NCC):
            j = i % 2
            pltpu.make_async_copy(
                tbl3_hbm.at[0, :, pl.ds(0, _CCH)], buf_v.at[j], wsems.at[j]
            ).wait()


@functools.partial(
    pl.kernel,
    mesh=_mesh,
    compiler_params=pltpu.CompilerParams(
        use_tc_tiling_on_sc=False, needs_layout_passes=False
    ),
    out_type=jax.ShapeDtypeStruct((EMBED, BATCH), jnp.float32),
    scratch_types=[
        pltpu.VMEM((BATCH,), jnp.int32),
        pltpu.VMEM((BATCH,), jnp.int32),
        pltpu.VMEM((BATCH,), jnp.int32),
        pltpu.VMEM((_NBUF, _CHUNK, 8), jnp.float32),
        pltpu.VMEM((BATCH,), jnp.float32),
        pltpu.VMEM((128,), jnp.float32),
        pltpu.SemaphoreType.DMA((_NBUF,)),
    ],
)
def _gather_kernel(
    idx_hbm, fa_hbm, fb_hbm, tail_hbm, out_hbm, idx_v, q_v, t_v, blk_v, col_v, tail_v, sems
):
    c = lax.axis_index("s") * _NC + lax.axis_index("c")
    pltpu.sync_copy(idx_hbm, idx_v)
    pltpu.sync_copy(tail_hbm.at[pl.ds(c * 128, 128)], tail_v)
    in_a = (c >> 3) < 2
    cbase = (c >> 3 & 1) * (PADROWS * 8) + (c & 7) * 128

    def compute(i, carry):
        r = idx_v[pl.ds(i * 16, 16)]
        e = (r >> 7) * 1024 + (r & 127) + cbase
        q_v[pl.ds(i * 16, 16)] = e >> 3
        t_v[pl.ds(i * 16, 16)] = e & 7
        return carry

    lax.fori_loop(0, BATCH // 16, compute, 0)

    def fire(k, j):
        idx_sl = q_v.at[pl.ds(k * _CHUNK, _CHUNK)]

        @pl.when(in_a)
        def _():
            pltpu.async_copy(fa_hbm.at[idx_sl], blk_v.at[j], sems.at[j])

        @pl.when(jnp.logical_not(in_a))
        def _():
            pltpu.async_copy(fb_hbm.at[idx_sl], blk_v.at[j], sems.at[j])

    def select(k, j):
        base = k * _CHUNK

        def body(i, carry):
            row = jax.lax.iota(jnp.int32, 16) + i * 16
            t16 = t_v[pl.ds(base + i * 16, 16)]
            vals = plsc.load_gather(blk_v.at[j], [row, t16])
            r16 = idx_v[pl.ds(base + i * 16, 16)]
            rt = jnp.minimum(jnp.maximum(r16 - RMAIN, 0), 127)
            tvals = plsc.load_gather(tail_v, [rt])
            vals = jnp.where(r16 >= RMAIN, tvals, vals)
            col_v[pl.ds(base + i * 16, 16)] = vals
            return carry

        lax.fori_loop(0, _CHUNK // 16, body, 0)

    for k in range(_NCHUNK):
        j = k % _NBUF
        if k >= _NBUF:
            pltpu.make_async_copy(fa_hbm.at[pl.ds(0, _CHUNK)], blk_v.at[j], sems.at[j]).wait()
            select(k - _NBUF, j)
        fire(k, j)
    for k in range(_NCHUNK - _NBUF, _NCHUNK):
        j = k % _NBUF
        pltpu.make_async_copy(fa_hbm.at[pl.ds(0, _CHUNK)], blk_v.at[j], sems.at[j]).wait()
        select(k, j)

    pltpu.sync_copy(col_v, out_hbm.at[c])


def kernel(indices, table):
    tbl_t = table.T
    tbl3 = tbl_t.reshape(4, 8, ROWS)
    flat_b = _sc_copy(tbl3)
    flat_a = _detile(tbl_t)
    tail = jnp.ravel(
        jnp.pad(table[RMAIN:], ((0, 127 - (ROWS - 1 - RMAIN)), (0, 0))).T.reshape(4, 8, 128)
    )
    fa2 = flat_a.reshape(2 * PADROWS, 8)
    fb2 = flat_b.reshape(2 * PADROWS, 8)
    out_t = _gather_kernel(indices.astype(jnp.int32), fa2, fb2, tail)
    return out_t.T


# speedup vs baseline: 18.7197x; 1.0161x over previous
import functools

import jax
import jax.numpy as jnp
from jax import lax
from jax.experimental import pallas as pl
from jax.experimental.pallas import tpu as pltpu
from jax.experimental.pallas import tpu_sc as plsc

ROWS = 1000001
PADROWS = 1000064
RMAIN = 999936
EMBED = 32
BATCH = 16384
_CW = 76928
_NK = PADROWS // _CW
_CHUNK = 512
_NCHUNK = BATCH // _CHUNK
_NBUF = 4
_SCW = 24
_TPW = RMAIN // 128 // 12
_CCH = 31 * 128
_NCC = _TPW // 31

_info = plsc.get_sparse_core_info()
_NC, _NS = _info.num_cores, _info.num_subcores

_mesh = plsc.VectorSubcoreMesh(core_axis_name="c", subcore_axis_name="s")


def _copy_body(in_ref, out_ref):
    blk = in_ref[...].reshape(8, _CW // 128, 128)
    out_ref[...] = jnp.transpose(blk, (1, 0, 2)).reshape(_CW * 8)


_detile = pl.pallas_call(
    _copy_body,
    grid=(2, _NK),
    in_specs=[pl.BlockSpec((8, _CW), lambda g, k: (g, k))],
    out_specs=pl.BlockSpec((_CW * 8,), lambda g, k: (g * _NK + k,)),
    out_shape=jax.ShapeDtypeStruct((2 * 8 * PADROWS,), jnp.float32),
)


@functools.partial(
    pl.kernel,
    mesh=_mesh,
    out_type=jax.ShapeDtypeStruct((2, RMAIN // 128 + 1, 8, 128), jnp.float32),
    scratch_types=[
        pltpu.VMEM((2, 31, 8, 128), jnp.float32),
        pltpu.SemaphoreType.DMA((2,)),
        pltpu.SemaphoreType.DMA((2,)),
    ],
)
def _sc_copy(tbl3_hbm, out_hbm, buf_v, rsems, wsems):
    w = lax.axis_index("s") * _NC + lax.axis_index("c")

    @pl.when(w < _SCW)
    def _():
        slab = 2 + w // 12
        tile0 = (w % 12) * _TPW
        for i in range(_NCC):
            j = i % 2
            b0 = tile0 + i * 31
            if i >= 2:
                pltpu.make_async_copy(
                    tbl3_hbm.at[0, :, pl.ds(0, _CCH)], buf_v.at[j], wsems.at[j]
                ).wait()
            for b in range(31):
                pltpu.async_copy(
                    tbl3_hbm.at[slab, :, pl.ds((b0 + b) * 128, 128)],
                    buf_v.at[j, b],
                    rsems.at[j],
                )
            pltpu.make_async_copy(
                tbl3_hbm.at[0, :, pl.ds(0, _CCH)], buf_v.at[j], rsems.at[j]
            ).wait()
            pltpu.async_copy(
                buf_v.at[j], out_hbm.at[slab - 2, pl.ds(b0, 31)], wsems.at[j]
            )
        for i in range(_NCC - 2, _NCC):
            j = i % 2
            pltpu.make_async_copy(
                tbl3_hbm.at[0, :, pl.ds(0, _CCH)], buf_v.at[j], wsems.at[j]
            ).wait()


@functools.partial(
    pl.kernel,
    mesh=_mesh,
    compiler_params=pltpu.CompilerParams(
        use_tc_tiling_on_sc=False, needs_layout_passes=False
    ),
    out_type=jax.ShapeDtypeStruct((EMBED, BATCH), jnp.float32),
    scratch_types=[
        pltpu.VMEM((BATCH,), jnp.int32),
        pltpu.VMEM((BATCH,), jnp.int32),
        pltpu.VMEM((BATCH,), jnp.int32),
        pltpu.VMEM((_NBUF, _CHUNK, 8), jnp.float32),
        pltpu.VMEM((BATCH,), jnp.float32),
        pltpu.VMEM((128,), jnp.float32),
        pltpu.SemaphoreType.DMA((_NBUF,)),
    ],
)
def _gather_kernel(
    idx_hbm, fa_hbm, fb_hbm, tail_hbm, out_hbm, idx_v, q_v, t_v, blk_v, col_v, tail_v, sems
):
    c = lax.axis_index("s") * _NC + lax.axis_index("c")
    pltpu.sync_copy(idx_hbm, idx_v)
    pltpu.sync_copy(tail_hbm.at[pl.ds(c * 128, 128)], tail_v)
    in_a = (c >> 3) < 2
    cbase = (c >> 3 & 1) * (PADROWS * 8) + (c & 7) * 128

    def fire(k, j):
        def compute(i, carry):
            o = k * _CHUNK + i * 16
            r = idx_v[pl.ds(o, 16)]
            e = (r >> 7) * 1024 + (r & 127) + cbase
            q_v[pl.ds(o, 16)] = e >> 3
            t_v[pl.ds(o, 16)] = e & 7
            return carry

        lax.fori_loop(0, _CHUNK // 16, compute, 0)
        idx_sl = q_v.at[pl.ds(k * _CHUNK, _CHUNK)]

        @pl.when(in_a)
        def _():
            pltpu.async_copy(fa_hbm.at[idx_sl], blk_v.at[j], sems.at[j])

        @pl.when(jnp.logical_not(in_a))
        def _():
            pltpu.async_copy(fb_hbm.at[idx_sl], blk_v.at[j], sems.at[j])

    def select(k, j):
        base = k * _CHUNK

        def body(i, carry):
            row = jax.lax.iota(jnp.int32, 16) + i * 16
            t16 = t_v[pl.ds(base + i * 16, 16)]
            vals = plsc.load_gather(blk_v.at[j], [row, t16])
            r16 = idx_v[pl.ds(base + i * 16, 16)]
            rt = jnp.minimum(jnp.maximum(r16 - RMAIN, 0), 127)
            tvals = plsc.load_gather(tail_v, [rt])
            vals = jnp.where(r16 >= RMAIN, tvals, vals)
            col_v[pl.ds(base + i * 16, 16)] = vals
            return carry

        lax.fori_loop(0, _CHUNK // 16, body, 0)

    for k in range(_NCHUNK):
        j = k % _NBUF
        if k >= _NBUF:
            pltpu.make_async_copy(fa_hbm.at[pl.ds(0, _CHUNK)], blk_v.at[j], sems.at[j]).wait()
            select(k - _NBUF, j)
        fire(k, j)
    for k in range(_NCHUNK - _NBUF, _NCHUNK):
        j = k % _NBUF
        pltpu.make_async_copy(fa_hbm.at[pl.ds(0, _CHUNK)], blk_v.at[j], sems.at[j]).wait()
        select(k, j)

    pltpu.sync_copy(col_v, out_hbm.at[c])


def kernel(indices, table):
    tbl_t = table.T
    tbl3 = tbl_t.reshape(4, 8, ROWS)
    flat_b = _sc_copy(tbl3)
    flat_a = _detile(tbl_t)
    tail = jnp.ravel(
        jnp.pad(table[RMAIN:], ((0, 127 - (ROWS - 1 - RMAIN)), (0, 0))).T.reshape(4, 8, 128)
    )
    fa2 = flat_a.reshape(2 * PADROWS, 8)
    fb2 = flat_b.reshape(2 * PADROWS, 8)
    out_t = _gather_kernel(indices.astype(jnp.int32), fa2, fb2, tail)
    return out_t.T


# speedup vs baseline: 19.0284x; 1.0165x over previous
import functools

import jax
import jax.numpy as jnp
from jax import lax
from jax.experimental import pallas as pl
from jax.experimental.pallas import tpu as pltpu
from jax.experimental.pallas import tpu_sc as plsc

ROWS = 1000001
PADROWS = 1000064
RMAIN = 999936
EMBED = 32
BATCH = 16384
_CW = 76928
_NK = PADROWS // _CW
_CHUNK = 1024
_NCHUNK = BATCH // _CHUNK
_NBUF = 4
_SCW = 24
_TPW = RMAIN // 128 // 12
_CCH = 31 * 128
_NCC = _TPW // 31

_info = plsc.get_sparse_core_info()
_NC, _NS = _info.num_cores, _info.num_subcores

_mesh = plsc.VectorSubcoreMesh(core_axis_name="c", subcore_axis_name="s")


def _copy_body(in_ref, out_ref):
    blk = in_ref[...].reshape(8, _CW // 128, 128)
    out_ref[...] = jnp.transpose(blk, (1, 0, 2)).reshape(_CW * 8)


_detile = pl.pallas_call(
    _copy_body,
    grid=(2, _NK),
    in_specs=[pl.BlockSpec((8, _CW), lambda g, k: (g, k))],
    out_specs=pl.BlockSpec((_CW * 8,), lambda g, k: (g * _NK + k,)),
    out_shape=jax.ShapeDtypeStruct((2 * 8 * PADROWS,), jnp.float32),
)


@functools.partial(
    pl.kernel,
    mesh=_mesh,
    out_type=jax.ShapeDtypeStruct((2, RMAIN // 128 + 1, 8, 128), jnp.float32),
    scratch_types=[
        pltpu.VMEM((2, 31, 8, 128), jnp.float32),
        pltpu.SemaphoreType.DMA((2,)),
        pltpu.SemaphoreType.DMA((2,)),
    ],
)
def _sc_copy(tbl3_hbm, out_hbm, buf_v, rsems, wsems):
    w = lax.axis_index("s") * _NC + lax.axis_index("c")

    @pl.when(w < _SCW)
    def _():
        slab = 2 + w // 12
        tile0 = (w % 12) * _TPW
        for i in range(_NCC):
            j = i % 2
            b0 = tile0 + i * 31
            if i >= 2:
                pltpu.make_async_copy(
                    tbl3_hbm.at[0, :, pl.ds(0, _CCH)], buf_v.at[j], wsems.at[j]
                ).wait()
            for b in range(31):
                pltpu.async_copy(
                    tbl3_hbm.at[slab, :, pl.ds((b0 + b) * 128, 128)],
                    buf_v.at[j, b],
                    rsems.at[j],
                )
            pltpu.make_async_copy(
                tbl3_hbm.at[0, :, pl.ds(0, _CCH)], buf_v.at[j], rsems.at[j]
            ).wait()
            pltpu.async_copy(
                buf_v.at[j], out_hbm.at[slab - 2, pl.ds(b0, 31)], wsems.at[j]
            )
        for i in range(_NCC - 2, _NCC):
            j = i % 2
            pltpu.make_async_copy(
                tbl3_hbm.at[0, :, pl.ds(0, _CCH)], buf_v.at[j], wsems.at[j]
            ).wait()


@functools.partial(
    pl.kernel,
    mesh=_mesh,
    compiler_params=pltpu.CompilerParams(
        use_tc_tiling_on_sc=False, needs_layout_passes=False
    ),
    out_type=jax.ShapeDtypeStruct((EMBED, BATCH), jnp.float32),
    scratch_types=[
        pltpu.VMEM((BATCH,), jnp.int32),
        pltpu.VMEM((BATCH,), jnp.int32),
        pltpu.VMEM((BATCH,), jnp.int32),
        pltpu.VMEM((_NBUF, _CHUNK, 8), jnp.float32),
        pltpu.VMEM((BATCH,), jnp.float32),
        pltpu.VMEM((128,), jnp.float32),
        pltpu.SemaphoreType.DMA((_NBUF,)),
    ],
)
def _gather_kernel(
    idx_hbm, fa_hbm, fb_hbm, tail_hbm, out_hbm, idx_v, q_v, t_v, blk_v, col_v, tail_v, sems
):
    c = lax.axis_index("s") * _NC + lax.axis_index("c")
    pltpu.sync_copy(idx_hbm, idx_v)
    pltpu.sync_copy(tail_hbm.at[pl.ds(c * 128, 128)], tail_v)
    in_a = (c >> 3) < 2
    cbase = (c >> 3 & 1) * (PADROWS * 8) + (c & 7) * 128

    def fire(k, j):
        def compute(i, carry):
            o = k * _CHUNK + i * 16
            r = idx_v[pl.ds(o, 16)]
            e = (r >> 7) * 1024 + (r & 127) + cbase
            q_v[pl.ds(o, 16)] = e >> 3
            t_v[pl.ds(o, 16)] = e & 7
            return carry

        lax.fori_loop(0, _CHUNK // 16, compute, 0)
        idx_sl = q_v.at[pl.ds(k * _CHUNK, _CHUNK)]

        @pl.when(in_a)
        def _():
            pltpu.async_copy(fa_hbm.at[idx_sl], blk_v.at[j], sems.at[j])

        @pl.when(jnp.logical_not(in_a))
        def _():
            pltpu.async_copy(fb_hbm.at[idx_sl], blk_v.at[j], sems.at[j])

    def select(k, j):
        base = k * _CHUNK

        def body(i, carry):
            row = jax.lax.iota(jnp.int32, 16) + i * 16
            t16 = t_v[pl.ds(base + i * 16, 16)]
            vals = plsc.load_gather(blk_v.at[j], [row, t16])
            r16 = idx_v[pl.ds(base + i * 16, 16)]
            rt = jnp.minimum(jnp.maximum(r16 - RMAIN, 0), 127)
            tvals = plsc.load_gather(tail_v, [rt])
            vals = jnp.where(r16 >= RMAIN, tvals, vals)
            col_v[pl.ds(base + i * 16, 16)] = vals
            return carry

        lax.fori_loop(0, _CHUNK // 16, body, 0)

    for k in range(_NCHUNK):
        j = k % _NBUF
        if k >= _NBUF:
            pltpu.make_async_copy(fa_hbm.at[pl.ds(0, _CHUNK)], blk_v.at[j], sems.at[j]).wait()
            select(k - _NBUF, j)
        fire(k, j)
    for k in range(_NCHUNK - _NBUF, _NCHUNK):
        j = k % _NBUF
        pltpu.make_async_copy(fa_hbm.at[pl.ds(0, _CHUNK)], blk_v.at[j], sems.at[j]).wait()
        select(k, j)

    pltpu.sync_copy(col_v, out_hbm.at[c])


def kernel(indices, table):
    tbl_t = table.T
    tbl3 = tbl_t.reshape(4, 8, ROWS)
    flat_b = _sc_copy(tbl3)
    flat_a = _detile(tbl_t)
    tail = jnp.ravel(
        jnp.pad(table[RMAIN:], ((0, 127 - (ROWS - 1 - RMAIN)), (0, 0))).T.reshape(4, 8, 128)
    )
    fa2 = flat_a.reshape(2 * PADROWS, 8)
    fb2 = flat_b.reshape(2 * PADROWS, 8)
    out_t = _gather_kernel(indices.astype(jnp.int32), fa2, fb2, tail)
    return out_t.T
